# baseline (speedup 1.0000x reference)
import functools

import jax
import jax.numpy as jnp
from jax import lax
from jax.experimental import pallas as pl
from jax.experimental.pallas import tpu as pltpu
from jax.experimental.pallas import tpu_sc as plsc

D_MODEL = 1024
SCALE = 32.0
CHUNK = 32
LANES = 16


def kernel(x, table, pos_enc):
    batch, seq = x.shape
    n_rows = batch * seq
    xf = x.reshape(n_rows).astype(jnp.int32)

    info = plsc.get_sparse_core_info()
    nc, ns = info.num_cores, info.num_subcores
    nw = nc * ns
    rows_per_w = n_rows // nw
    n_chunks = rows_per_w // CHUNK

    mesh = plsc.VectorSubcoreMesh(core_axis_name="c", subcore_axis_name="s")

    @functools.partial(
        pl.kernel,
        mesh=mesh,
        out_type=jax.ShapeDtypeStruct((n_rows, D_MODEL), jnp.float32),
        scratch_types=[
            pltpu.VMEM((CHUNK,), jnp.int32),
            pltpu.VMEM((CHUNK, D_MODEL), jnp.float32),
            pltpu.VMEM((CHUNK, D_MODEL), jnp.float32),
            pltpu.SemaphoreType.DMA,
        ],
    )
    def emb_kernel(x_hbm, tab_hbm, pos_hbm, out_hbm, idx_v, rows_v, pos_v, sem):
        wid = lax.axis_index("s") * nc + lax.axis_index("c")
        base = wid * rows_per_w

        def chunk_body(ci, _):
            f = pl.multiple_of(base + ci * CHUNK, CHUNK)
            p = pl.multiple_of(lax.rem(f, seq), CHUNK)
            pltpu.sync_copy(x_hbm.at[pl.ds(f, CHUNK)], idx_v)
            gather = pltpu.async_copy(tab_hbm.at[idx_v], rows_v, sem)
            pltpu.sync_copy(pos_hbm.at[pl.ds(p, CHUNK)], pos_v)
            gather.wait()

            def row_body(r, _):
                def col_body(c, _):
                    sl = pl.ds(c * LANES, LANES)
                    rows_v[r, sl] = rows_v[r, sl] * SCALE + pos_v[r, sl]
                    return 0

                return lax.fori_loop(0, D_MODEL // LANES, col_body, 0)

            lax.fori_loop(0, CHUNK, row_body, 0)
            pltpu.sync_copy(rows_v, out_hbm.at[pl.ds(f, CHUNK)])
            return 0

        lax.fori_loop(0, n_chunks, chunk_body, 0)

    out = emb_kernel(xf, table, pos_enc)
    return out.reshape(batch, seq, D_MODEL)


# speedup vs baseline: 1.2116x; 1.2116x over previous
import functools

import jax
import jax.numpy as jnp
from jax import lax
from jax.experimental import pallas as pl
from jax.experimental.pallas import tpu as pltpu
from jax.experimental.pallas import tpu_sc as plsc

D_MODEL = 1024
SCALE = 32.0
CH = 16
LANES = 16
NBUF = 4
UNROLL = 4


def kernel(x, table, pos_enc):
    batch, seq = x.shape
    n_rows = batch * seq
    xf = x.reshape(n_rows).astype(jnp.int32)

    info = plsc.get_sparse_core_info()
    nc, ns = info.num_cores, info.num_subcores
    nw = nc * ns
    seq_per_w = seq // nw
    sub_per_w = seq_per_w // CH
    n_chunks = batch * sub_per_w

    mesh = plsc.VectorSubcoreMesh(core_axis_name="c", subcore_axis_name="s")

    @functools.partial(
        pl.kernel,
        mesh=mesh,
        out_type=jax.ShapeDtypeStruct((n_rows, D_MODEL), jnp.float32),
        scratch_types=[
            pltpu.VMEM((2, CH, D_MODEL), jnp.float32),
            pltpu.VMEM((NBUF, CH, D_MODEL), jnp.float32),
            pltpu.VMEM((NBUF, CH), jnp.int32),
        ]
        + [pltpu.SemaphoreType.DMA] * (2 * NBUF + 2),
    )
    def emb_kernel(x_hbm, tab_hbm, pos_hbm, out_hbm, pos_v, rows_v, idx_v,
                   *sems):
        gsem = sems[:NBUF]
        wsem = sems[NBUF:2 * NBUF]
        psem = sems[2 * NBUF:]
        wid = lax.axis_index("c") * ns + lax.axis_index("s")
        s0 = wid * seq_per_w

        def flat_base(ci):
            sub, b = divmod(ci, batch)
            return pl.multiple_of(b * seq + s0 + sub * CH, CH)

        def start_pos(sub):
            pb = sub % 2
            off = pl.multiple_of(s0 + sub * CH, CH)
            return pltpu.async_copy(
                pos_hbm.at[pl.ds(off, CH)], pos_v.at[pb], psem[pb])

        def start_gather(ci):
            bi = ci % NBUF
            f = flat_base(ci)
            pltpu.sync_copy(x_hbm.at[pl.ds(f, CH)], idx_v.at[bi])
            return pltpu.async_copy(
                tab_hbm.at[idx_v.at[bi]], rows_v.at[bi], gsem[bi])

        pos_h = {0: start_pos(0)}
        g_h = {0: start_gather(0), 1: start_gather(1)}
        wb_h = {}

        for ci in range(n_chunks):
            sub, b = divmod(ci, batch)
            bi = ci % NBUF
            if b == 0 and sub + 1 < sub_per_w:
                pos_h[sub + 1] = start_pos(sub + 1)
            if ci + 2 < n_chunks:
                if ci - 2 >= 0:
                    wb_h.pop(ci - 2).wait()
                g_h[ci + 2] = start_gather(ci + 2)
            if b == 0:
                pos_h.pop(sub).wait()
            g_h.pop(ci).wait()

            rv = rows_v.at[bi]
            pv = pos_v.at[sub % 2]

            def row_body(r, _, rv=rv, pv=pv):
                def col_body(c, _):
                    for u in range(UNROLL):
                        sl = pl.ds((c * UNROLL + u) * LANES, LANES)
                        rv[r, sl] = rv[r, sl] * SCALE + pv[r, sl]
                    return 0

                return lax.fori_loop(0, D_MODEL // (LANES * UNROLL),
                                     col_body, 0)

            lax.fori_loop(0, CH, row_body, 0)
            wb_h[ci] = pltpu.async_copy(
                rv, out_hbm.at[pl.ds(flat_base(ci), CH)], wsem[bi])

        for ci in sorted(wb_h):
            wb_h.pop(ci).wait()

    out = emb_kernel(xf, table, pos_enc)
    return out.reshape(batch, seq, D_MODEL)


# speedup vs baseline: 2.9455x; 2.4311x over previous
import functools

import jax
import jax.numpy as jnp
from jax import lax
from jax.experimental import pallas as pl
from jax.experimental.pallas import tpu as pltpu
from jax.experimental.pallas import tpu_sc as plsc

D_MODEL = 1024
SCALE = 32.0
CH = 16
LANES = 16
NBUF = 4
UNROLL = 4


def kernel(x, table, pos_enc):
    batch, seq = x.shape
    n_rows = batch * seq
    xf = x.reshape(n_rows).astype(jnp.int32)

    info = plsc.get_sparse_core_info()
    nc, ns = info.num_cores, info.num_subcores
    nw = nc * ns
    seq_per_w = seq // nw
    sub_per_w = seq_per_w // CH
    n_chunks = batch * sub_per_w

    mesh = plsc.VectorSubcoreMesh(core_axis_name="c", subcore_axis_name="s")

    @functools.partial(
        pl.kernel,
        mesh=mesh,
        out_type=jax.ShapeDtypeStruct((n_rows, D_MODEL), jnp.float32),
        scratch_types=[
            pltpu.VMEM((2, CH, D_MODEL), jnp.float32),
            pltpu.VMEM((NBUF, CH, D_MODEL), jnp.float32),
            pltpu.VMEM((NBUF, CH), jnp.int32),
        ]
        + [pltpu.SemaphoreType.DMA] * (2 * NBUF + 2),
    )
    def emb_kernel(x_hbm, tab_hbm, pos_hbm, out_hbm, pos_v, rows_v, idx_v,
                   *sems):
        gsem = sems[:NBUF]
        wsem = sems[NBUF:2 * NBUF]
        psem = sems[2 * NBUF:]
        wid = lax.axis_index("c") * ns + lax.axis_index("s")
        s0 = wid * seq_per_w

        def flat_base(ci):
            sub, b = divmod(ci, batch)
            return pl.multiple_of(b * seq + s0 + sub * CH, CH)

        def start_pos(sub):
            pb = sub % 2
            off = pl.multiple_of(s0 + sub * CH, CH)
            return pltpu.async_copy(
                pos_hbm.at[pl.ds(off, CH)], pos_v.at[pb], psem[pb])

        def start_gather(ci):
            bi = ci % NBUF
            f = flat_base(ci)
            pltpu.sync_copy(x_hbm.at[pl.ds(f, CH)], idx_v.at[bi])
            return pltpu.async_copy(
                tab_hbm.at[idx_v.at[bi]], rows_v.at[bi], gsem[bi])

        pos_h = {0: start_pos(0)}
        g_h = {0: start_gather(0), 1: start_gather(1)}
        wb_h = {}

        for ci in range(n_chunks):
            sub, b = divmod(ci, batch)
            bi = ci % NBUF
            if b == 0 and sub + 1 < sub_per_w:
                pos_h[sub + 1] = start_pos(sub + 1)
            if ci + 2 < n_chunks:
                if ci - 2 >= 0:
                    wb_h.pop(ci - 2).wait()
                g_h[ci + 2] = start_gather(ci + 2)
            if b == 0:
                pos_h.pop(sub).wait()
            g_h.pop(ci).wait()

            rv = rows_v.at[bi]
            pv = pos_v.at[sub % 2]

            def row_body(r, _, rv=rv, pv=pv):
                @plsc.parallel_loop(0, D_MODEL, step=LANES, unroll=UNROLL)
                def _col(c):
                    sl = pl.ds(c, LANES)
                    rv[r, sl] = rv[r, sl] * SCALE + pv[r, sl]

                return 0

            lax.fori_loop(0, CH, row_body, 0)
            wb_h[ci] = pltpu.async_copy(
                rv, out_hbm.at[pl.ds(flat_base(ci), CH)], wsem[bi])

        for ci in sorted(wb_h):
            wb_h.pop(ci).wait()

    out = emb_kernel(xf, table, pos_enc)
    return out.reshape(batch, seq, D_MODEL)


# speedup vs baseline: 3.0366x; 1.0309x over previous
import functools

import jax
import jax.numpy as jnp
from jax import lax
from jax.experimental import pallas as pl
from jax.experimental.pallas import tpu as pltpu
from jax.experimental.pallas import tpu_sc as plsc

D_MODEL = 1024
SCALE = 32.0
CH = 16
LANES = 16
NBUF = 4
UNROLL = 8


def kernel(x, table, pos_enc):
    batch, seq = x.shape
    n_rows = batch * seq
    xf = x.reshape(n_rows).astype(jnp.int32)

    info = plsc.get_sparse_core_info()
    nc, ns = info.num_cores, info.num_subcores
    nw = nc * ns
    seq_per_w = seq // nw
    sub_per_w = seq_per_w // CH
    n_chunks = batch * sub_per_w

    mesh = plsc.VectorSubcoreMesh(core_axis_name="c", subcore_axis_name="s")

    @functools.partial(
        pl.kernel,
        mesh=mesh,
        out_type=jax.ShapeDtypeStruct((n_rows, D_MODEL), jnp.float32),
        scratch_types=[
            pltpu.VMEM((2, CH, D_MODEL), jnp.float32),
            pltpu.VMEM((NBUF, CH, D_MODEL), jnp.float32),
            pltpu.VMEM((batch, seq_per_w), jnp.int32),
        ]
        + [pltpu.SemaphoreType.DMA] * (2 * NBUF + 3),
    )
    def emb_kernel(x_hbm, tab_hbm, pos_hbm, out_hbm, pos_v, rows_v, idx_v,
                   *sems):
        gsem = sems[:NBUF]
        wsem = sems[NBUF:2 * NBUF]
        psem = sems[2 * NBUF:2 * NBUF + 2]
        isem = sems[2 * NBUF + 2]
        wid = lax.axis_index("c") * ns + lax.axis_index("s")
        s0 = wid * seq_per_w

        idx_hs = []
        for b in range(batch):
            off = pl.multiple_of(b * seq + s0, CH)
            idx_hs.append(pltpu.async_copy(
                x_hbm.at[pl.ds(off, seq_per_w)], idx_v.at[b], isem))
        for h in idx_hs:
            h.wait()

        def flat_base(ci):
            sub, b = divmod(ci, batch)
            return pl.multiple_of(b * seq + s0 + sub * CH, CH)

        def start_pos(sub):
            pb = sub % 2
            off = pl.multiple_of(s0 + sub * CH, CH)
            return pltpu.async_copy(
                pos_hbm.at[pl.ds(off, CH)], pos_v.at[pb], psem[pb])

        def start_gather(ci):
            bi = ci % NBUF
            sub, b = divmod(ci, batch)
            idx_ref = idx_v.at[b, pl.ds(sub * CH, CH)]
            return pltpu.async_copy(
                tab_hbm.at[idx_ref], rows_v.at[bi], gsem[bi])

        pos_h = {0: start_pos(0)}
        g_h = {0: start_gather(0), 1: start_gather(1)}
        wb_h = {}

        for ci in range(n_chunks):
            sub, b = divmod(ci, batch)
            bi = ci % NBUF
            if b == 0 and sub + 1 < sub_per_w:
                pos_h[sub + 1] = start_pos(sub + 1)
            if ci + 2 < n_chunks:
                if ci - 2 >= 0:
                    wb_h.pop(ci - 2).wait()
                g_h[ci + 2] = start_gather(ci + 2)
            if b == 0:
                pos_h.pop(sub).wait()
            g_h.pop(ci).wait()

            rv = rows_v.at[bi]
            pv = pos_v.at[sub % 2]

            def row_body(r, _, rv=rv, pv=pv):
                @plsc.parallel_loop(0, D_MODEL, step=LANES, unroll=UNROLL)
                def _col(c):
                    sl = pl.ds(c, LANES)
                    rv[r, sl] = rv[r, sl] * SCALE + pv[r, sl]

                return 0

            lax.fori_loop(0, CH, row_body, 0)
            wb_h[ci] = pltpu.async_copy(
                rv, out_hbm.at[pl.ds(flat_base(ci), CH)], wsem[bi])

        for ci in sorted(wb_h):
            wb_h.pop(ci).wait()

    out = emb_kernel(xf, table, pos_enc)
    return out.reshape(batch, seq, D_MODEL)


# speedup vs baseline: 3.1288x; 1.0304x over previous
import functools

import jax
import jax.numpy as jnp
from jax import lax
from jax.experimental import pallas as pl
from jax.experimental.pallas import tpu as pltpu
from jax.experimental.pallas import tpu_sc as plsc

D_MODEL = 1024
SCALE = 32.0
CH = 16
LANES = 16
NBUF = 4
UNROLL = 8


def kernel(x, table, pos_enc):
    batch, seq = x.shape
    n_rows = batch * seq
    xf = x.reshape(n_rows).astype(jnp.int32)

    info = plsc.get_sparse_core_info()
    nc, ns = info.num_cores, info.num_subcores
    nw = nc * ns
    seq_per_w = seq // nw
    sub_per_w = seq_per_w // CH
    n_chunks = batch * sub_per_w
    n_rounds = sub_per_w

    mesh = plsc.VectorSubcoreMesh(core_axis_name="c", subcore_axis_name="s")

    @functools.partial(
        pl.kernel,
        mesh=mesh,
        out_type=jax.ShapeDtypeStruct((n_rows, D_MODEL), jnp.float32),
        scratch_types=[
            pltpu.VMEM((2, CH, D_MODEL), jnp.float32),
            pltpu.VMEM((NBUF, CH, D_MODEL), jnp.float32),
            pltpu.VMEM((batch, seq_per_w), jnp.int32),
        ]
        + [pltpu.SemaphoreType.DMA] * (2 * NBUF + 3),
    )
    def emb_kernel(x_hbm, tab_hbm, pos_hbm, out_hbm, pos_v, rows_v, idx_v,
                   *sems):
        gsem = sems[:NBUF]
        wsem = sems[NBUF:2 * NBUF]
        psem = sems[2 * NBUF:2 * NBUF + 2]
        isem = sems[2 * NBUF + 2]
        wid = lax.axis_index("c") * ns + lax.axis_index("s")
        s0 = wid * seq_per_w

        for b in range(batch):
            off = pl.multiple_of(b * seq + s0, CH)
            pltpu.async_copy(x_hbm.at[pl.ds(off, seq_per_w)], idx_v.at[b],
                             isem)
        for b in range(batch):
            pltpu.make_async_copy(x_hbm.at[pl.ds(0, seq_per_w)],
                                  idx_v.at[b], isem).wait()

        def out_slice(ci):
            sub = lax.div(ci, batch)
            b = lax.rem(ci, batch)
            f = pl.multiple_of(b * seq + s0 + sub * CH, CH)
            return out_hbm.at[pl.ds(f, CH)]

        def idx_slice(ci):
            sub = lax.div(ci, batch)
            b = lax.rem(ci, batch)
            return idx_v.at[b, pl.ds(sub * CH, CH)]

        def start_pos(sub, pb):
            off = pl.multiple_of(s0 + sub * CH, CH)
            pltpu.async_copy(pos_hbm.at[pl.ds(off, CH)], pos_v.at[pb],
                             psem[pb])

        def start_gather(ci, bi):
            pltpu.async_copy(tab_hbm.at[idx_slice(ci)], rows_v.at[bi],
                             gsem[bi])

        def wait_gather(ci, bi):
            pltpu.make_async_copy(tab_hbm.at[idx_slice(ci)], rows_v.at[bi],
                                  gsem[bi]).wait()

        def wait_wb(ci, bi):
            pltpu.make_async_copy(rows_v.at[bi], out_slice(ci),
                                  wsem[bi]).wait()

        def wait_pos(pb):
            pltpu.make_async_copy(pos_hbm.at[pl.ds(0, CH)], pos_v.at[pb],
                                  psem[pb]).wait()

        start_pos(0, 0)
        start_gather(0, 0)
        start_gather(1, 1)

        def maybe_when(cond, fn):
            if isinstance(cond, bool):
                if cond:
                    fn()
            else:
                pl.when(cond)(fn)

        def round_body(outer, _):
            for h in range(2):
                r = outer * 2 + h
                not_first = (outer > 0) if h == 0 else True
                not_last = True if h == 0 else (outer < 1)
                ci0 = r * batch
                for k in range(batch):
                    ci = ci0 + k
                    bi = k
                    nb = (k + 2) % NBUF
                    if k < 2:
                        maybe_when(not_first, lambda ci=ci, nb=nb:
                                   wait_wb(ci - 2, nb))
                        start_gather(ci + 2, nb)
                    else:
                        def drain_and_gather(ci=ci, nb=nb):
                            wait_wb(ci - 2, nb)
                            start_gather(ci + 2, nb)

                        maybe_when(not_last, drain_and_gather)
                    if k == 0:
                        wait_pos(h)
                        maybe_when(not_last, lambda r=r, h=h:
                                   start_pos(r + 1, (h + 1) % 2))
                    wait_gather(ci, bi)

                    rv = rows_v.at[bi]
                    pv = pos_v.at[h]

                    def row_body(rr, _, rv=rv, pv=pv):
                        @plsc.parallel_loop(0, D_MODEL, step=LANES,
                                            unroll=UNROLL)
                        def _col(c):
                            sl = pl.ds(c, LANES)
                            rv[rr, sl] = rv[rr, sl] * SCALE + pv[rr, sl]

                        return 0

                    lax.fori_loop(0, CH, row_body, 0)
                    pltpu.async_copy(rv, out_slice(ci), wsem[bi])
            return 0

        lax.fori_loop(0, n_rounds // 2, round_body, 0)
        last = n_chunks - batch
        for k in range(batch):
            wait_wb(last + k, k)

    out = emb_kernel(xf, table, pos_enc)
    return out.reshape(batch, seq, D_MODEL)


# speedup vs baseline: 3.1620x; 1.0106x over previous
import functools

import jax
import jax.numpy as jnp
from jax import lax
from jax.experimental import pallas as pl
from jax.experimental.pallas import tpu as pltpu
from jax.experimental.pallas import tpu_sc as plsc

D_MODEL = 1024
SCALE = 32.0
CH = 16
LANES = 16
NBUF = 4
UNROLL = 4


def kernel(x, table, pos_enc):
    batch, seq = x.shape
    n_rows = batch * seq
    xf = x.reshape(n_rows).astype(jnp.int32)

    info = plsc.get_sparse_core_info()
    nc, ns = info.num_cores, info.num_subcores
    nw = nc * ns
    seq_per_w = seq // nw
    sub_per_w = seq_per_w // CH
    n_chunks = batch * sub_per_w
    n_rounds = sub_per_w

    mesh = plsc.VectorSubcoreMesh(core_axis_name="c", subcore_axis_name="s")

    @functools.partial(
        pl.kernel,
        mesh=mesh,
        out_type=jax.ShapeDtypeStruct((n_rows, D_MODEL), jnp.float32),
        scratch_types=[
            pltpu.VMEM((2, CH, D_MODEL), jnp.float32),
            pltpu.VMEM((NBUF, CH, D_MODEL), jnp.float32),
            pltpu.VMEM((batch, seq_per_w), jnp.int32),
        ]
        + [pltpu.SemaphoreType.DMA] * (2 * NBUF + 3),
    )
    def emb_kernel(x_hbm, tab_hbm, pos_hbm, out_hbm, pos_v, rows_v, idx_v,
                   *sems):
        gsem = sems[:NBUF]
        wsem = sems[NBUF:2 * NBUF]
        psem = sems[2 * NBUF:2 * NBUF + 2]
        isem = sems[2 * NBUF + 2]
        wid = lax.axis_index("c") * ns + lax.axis_index("s")
        s0 = wid * seq_per_w

        for b in range(batch):
            off = pl.multiple_of(b * seq + s0, CH)
            pltpu.async_copy(x_hbm.at[pl.ds(off, seq_per_w)], idx_v.at[b],
                             isem)
        for b in range(batch):
            pltpu.make_async_copy(x_hbm.at[pl.ds(0, seq_per_w)],
                                  idx_v.at[b], isem).wait()

        def out_slice(ci):
            sub = lax.div(ci, batch)
            b = lax.rem(ci, batch)
            f = pl.multiple_of(b * seq + s0 + sub * CH, CH)
            return out_hbm.at[pl.ds(f, CH)]

        def idx_slice(ci):
            sub = lax.div(ci, batch)
            b = lax.rem(ci, batch)
            return idx_v.at[b, pl.ds(sub * CH, CH)]

        def start_pos(sub, pb):
            off = pl.multiple_of(s0 + sub * CH, CH)
            pltpu.async_copy(pos_hbm.at[pl.ds(off, CH)], pos_v.at[pb],
                             psem[pb])

        def start_gather(ci, bi):
            pltpu.async_copy(tab_hbm.at[idx_slice(ci)], rows_v.at[bi],
                             gsem[bi])

        def wait_gather(ci, bi):
            pltpu.make_async_copy(tab_hbm.at[idx_slice(ci)], rows_v.at[bi],
                                  gsem[bi]).wait()

        def wait_wb(ci, bi):
            pltpu.make_async_copy(rows_v.at[bi], out_slice(ci),
                                  wsem[bi]).wait()

        def wait_pos(pb):
            pltpu.make_async_copy(pos_hbm.at[pl.ds(0, CH)], pos_v.at[pb],
                                  psem[pb]).wait()

        start_pos(0, 0)
        start_gather(0, 0)
        start_gather(1, 1)

        def maybe_when(cond, fn):
            if isinstance(cond, bool):
                if cond:
                    fn()
            else:
                pl.when(cond)(fn)

        def round_body(outer, _):
            for h in range(2):
                r = outer * 2 + h
                not_first = (outer > 0) if h == 0 else True
                not_last = True if h == 0 else (outer < 1)
                ci0 = r * batch
                for k in range(batch):
                    ci = ci0 + k
                    bi = k
                    nb = (k + 2) % NBUF
                    if k < 2:
                        maybe_when(not_first, lambda ci=ci, nb=nb:
                                   wait_wb(ci - 2, nb))
                        start_gather(ci + 2, nb)
                    else:
                        def drain_and_gather(ci=ci, nb=nb):
                            wait_wb(ci - 2, nb)
                            start_gather(ci + 2, nb)

                        maybe_when(not_last, drain_and_gather)
                    if k == 0:
                        wait_pos(h)
                        maybe_when(not_last, lambda r=r, h=h:
                                   start_pos(r + 1, (h + 1) % 2))
                    wait_gather(ci, bi)

                    rv = rows_v.at[bi]
                    pv = pos_v.at[h]

                    def row_body(rr, _, rv=rv, pv=pv):
                        @plsc.parallel_loop(0, D_MODEL, step=LANES,
                                            unroll=UNROLL)
                        def _col(c):
                            sl = pl.ds(c, LANES)
                            rv[rr, sl] = rv[rr, sl] * SCALE + pv[rr, sl]

                        return 0

                    lax.fori_loop(0, CH, row_body, 0)
                    pltpu.async_copy(rv, out_slice(ci), wsem[bi])
            return 0

        lax.fori_loop(0, n_rounds // 2, round_body, 0)
        last = n_chunks - batch
        for k in range(batch):
            wait_wb(last + k, k)

    out = emb_kernel(xf, table, pos_enc)
    return out.reshape(batch, seq, D_MODEL)


# speedup vs baseline: 3.2628x; 1.0319x over previous
import functools

import jax
import jax.numpy as jnp
from jax import lax
from jax.experimental import pallas as pl
from jax.experimental.pallas import tpu as pltpu
from jax.experimental.pallas import tpu_sc as plsc

D_MODEL = 1024
SCALE = 32.0
CH = 8
LANES = 16
NBUF = 8
LOOK = 4
UNROLL = 4


def kernel(x, table, pos_enc):
    batch, seq = x.shape
    n_rows = batch * seq
    xf = x.reshape(n_rows).astype(jnp.int32)

    info = plsc.get_sparse_core_info()
    nc, ns = info.num_cores, info.num_subcores
    nw = nc * ns
    seq_per_w = seq // nw
    sub_per_w = seq_per_w // CH
    n_chunks = batch * sub_per_w
    n_rounds = sub_per_w
    n_outer = n_rounds // 2

    mesh = plsc.VectorSubcoreMesh(core_axis_name="c", subcore_axis_name="s")

    @functools.partial(
        pl.kernel,
        mesh=mesh,
        out_type=jax.ShapeDtypeStruct((n_rows, D_MODEL), jnp.float32),
        scratch_types=[
            pltpu.VMEM((2, CH, D_MODEL), jnp.float32),
            pltpu.VMEM((NBUF, CH, D_MODEL), jnp.float32),
            pltpu.VMEM((batch, seq_per_w), jnp.int32),
        ]
        + [pltpu.SemaphoreType.DMA] * (2 * NBUF + 3),
    )
    def emb_kernel(x_hbm, tab_hbm, pos_hbm, out_hbm, pos_v, rows_v, idx_v,
                   *sems):
        gsem = sems[:NBUF]
        wsem = sems[NBUF:2 * NBUF]
        psem = sems[2 * NBUF:2 * NBUF + 2]
        isem = sems[2 * NBUF + 2]
        wid = lax.axis_index("c") * ns + lax.axis_index("s")
        s0 = wid * seq_per_w

        for b in range(batch):
            off = pl.multiple_of(b * seq + s0, CH)
            pltpu.async_copy(x_hbm.at[pl.ds(off, seq_per_w)], idx_v.at[b],
                             isem)
        for b in range(batch):
            pltpu.make_async_copy(x_hbm.at[pl.ds(0, seq_per_w)],
                                  idx_v.at[b], isem).wait()

        def out_slice(ci):
            sub = lax.div(ci, batch)
            b = lax.rem(ci, batch)
            f = pl.multiple_of(b * seq + s0 + sub * CH, CH)
            return out_hbm.at[pl.ds(f, CH)]

        def idx_slice(ci):
            sub = lax.div(ci, batch)
            b = lax.rem(ci, batch)
            return idx_v.at[b, pl.ds(sub * CH, CH)]

        def start_pos(sub, pb):
            off = pl.multiple_of(s0 + sub * CH, CH)
            pltpu.async_copy(pos_hbm.at[pl.ds(off, CH)], pos_v.at[pb],
                             psem[pb])

        def start_gather(ci, bi):
            pltpu.async_copy(tab_hbm.at[idx_slice(ci)], rows_v.at[bi],
                             gsem[bi])

        def wait_gather(ci, bi):
            pltpu.make_async_copy(tab_hbm.at[idx_slice(ci)], rows_v.at[bi],
                                  gsem[bi]).wait()

        def wait_wb(ci, bi):
            pltpu.make_async_copy(rows_v.at[bi], out_slice(ci),
                                  wsem[bi]).wait()

        def wait_pos(pb):
            pltpu.make_async_copy(pos_hbm.at[pl.ds(0, CH)], pos_v.at[pb],
                                  psem[pb]).wait()

        def maybe_when(cond, fn):
            if isinstance(cond, bool):
                if cond:
                    fn()
            else:
                pl.when(cond)(fn)

        start_pos(0, 0)
        for ci in range(LOOK):
            start_gather(ci, ci)

        def round_body(outer, _):
            for h in range(2):
                r = outer * 2 + h
                ci0 = r * batch
                for k in range(batch):
                    ci = ci0 + k
                    bi = h * batch + k
                    nb = (bi + LOOK) % NBUF
                    gather_ok = True if h == 0 else (outer < n_outer - 1)
                    drain_ok = (outer > 0) if h == 0 else True

                    def drain(ci=ci, nb=nb):
                        wait_wb(ci - LOOK, nb)

                    def drain_and_gather(ci=ci, nb=nb, do_drain=drain_ok):
                        maybe_when(do_drain, lambda: wait_wb(ci - LOOK, nb))
                        start_gather(ci + LOOK, nb)

                    if isinstance(gather_ok, bool):
                        if gather_ok:
                            maybe_when(drain_ok, drain)
                            start_gather(ci + LOOK, nb)
                    else:
                        maybe_when(gather_ok, lambda ci=ci, nb=nb: (
                            wait_wb(ci - LOOK, nb),
                            start_gather(ci + LOOK, nb))[-1])
                    if k == 0:
                        wait_pos(h)
                        pos_ok = True if h == 0 else (outer < n_outer - 1)
                        maybe_when(pos_ok, lambda r=r, h=h:
                                   start_pos(r + 1, (h + 1) % 2))
                    wait_gather(ci, bi)

                    rv = rows_v.at[bi]
                    pv = pos_v.at[h]

                    def row_body(rr, _, rv=rv, pv=pv):
                        @plsc.parallel_loop(0, D_MODEL, step=LANES,
                                            unroll=UNROLL)
                        def _col(c):
                            sl = pl.ds(c, LANES)
                            rv[rr, sl] = rv[rr, sl] * SCALE + pv[rr, sl]

                        return 0

                    lax.fori_loop(0, CH, row_body, 0)
                    pltpu.async_copy(rv, out_slice(ci), wsem[bi])
            return 0

        lax.fori_loop(0, n_outer, round_body, 0)
        last = n_chunks - NBUF
        for j in range(NBUF):
            wait_wb(last + j, j)

    out = emb_kernel(xf, table, pos_enc)
    return out.reshape(batch, seq, D_MODEL)
